# baseline (speedup 1.0000x reference)
import jax
import jax.numpy as jnp
import numpy as np
from jax import lax
from jax.experimental import pallas as pl
from jax.experimental.pallas import tpu as pltpu
from jax.experimental.pallas import tpu_sc as plsc

_N = 36
_PERM = tuple(int(v) for v in np.random.RandomState(0).permutation(_N))

_B = 4096
_D = 128
_NC = 2
_NS = 16
_NW = _NC * _NS
_BPW = _B // _NW


_NBUF = 4


def _body(x_hbm, out_hbm, bufs, semr, semw):
    wid = lax.axis_index("s") * _NC + lax.axis_index("c")
    b0 = wid * _BPW

    def start_in(j, b):
        pltpu.async_copy(
            x_hbm.at[_PERM[j], pl.ds(b0, _BPW), :], bufs[b], semr[b]
        )

    def wait_in(j, b):
        pltpu.make_async_copy(
            x_hbm.at[_PERM[j], pl.ds(b0, _BPW), :], bufs[b], semr[b]
        ).wait()

    def start_out(j, b):
        pltpu.async_copy(bufs[b], out_hbm.at[j, pl.ds(b0, _BPW), :], semw[b])

    def wait_out(j, b):
        pltpu.make_async_copy(
            bufs[b], out_hbm.at[j, pl.ds(b0, _BPW), :], semw[b]
        ).wait()

    for b in range(_NBUF):
        start_in(b, b)

    for j in range(_N):
        b = j % _NBUF
        wait_in(j, b)
        start_out(j, b)
        r = j + 1
        if _NBUF <= r < _N:
            rb = r % _NBUF
            wait_out(r - _NBUF, rb)
            start_in(r, rb)

    for j in range(_N - _NBUF, _N):
        wait_out(j, j % _NBUF)


@jax.jit
def kernel(x):
    xt = jnp.transpose(x, (1, 0, 2))
    mesh = plsc.VectorSubcoreMesh(core_axis_name="c", subcore_axis_name="s")
    out_t = pl.kernel(
        _body,
        out_type=jax.ShapeDtypeStruct((_N, _B, _D), x.dtype),
        mesh=mesh,
        scratch_types=[
            [pltpu.VMEM((_BPW, _D), jnp.float32) for _ in range(_NBUF)],
            [pltpu.SemaphoreType.DMA for _ in range(_NBUF)],
            [pltpu.SemaphoreType.DMA for _ in range(_NBUF)],
        ],
    )(xt)
    return jnp.transpose(out_t, (1, 0, 2))


# speedup vs baseline: 1.2042x; 1.2042x over previous
import jax
import jax.numpy as jnp
import numpy as np
from jax import lax
from jax.experimental import pallas as pl
from jax.experimental.pallas import tpu as pltpu
from jax.experimental.pallas import tpu_sc as plsc

_N = 36
_PERM = tuple(int(v) for v in np.random.RandomState(0).permutation(_N))

_B = 4096
_D = 128
_NC = 2
_NS = 16
_NW = _NC * _NS
_BPW = _B // _NW


_WIN = 256
_HALF = _N // 2


def _half_body(x_hbm, out_hbm, bufs, sems, b0, j0):
    def start_in(j, b):
        pltpu.async_copy(
            x_hbm.at[_PERM[j0 + j], pl.ds(b0, _WIN), :], bufs[b], sems[b]
        )

    def wait_in(j, b):
        pltpu.make_async_copy(
            x_hbm.at[_PERM[j0 + j], pl.ds(b0, _WIN), :], bufs[b], sems[b]
        ).wait()

    start_in(0, 0)
    start_in(1, 1)
    for j in range(_HALF):
        b = j % 2
        wait_in(j, b)
        pltpu.sync_copy(bufs[b], out_hbm.at[j0 + j, pl.ds(b0, _WIN), :])
        if j + 2 < _HALF:
            start_in(j + 2, b)


def _body(x_hbm, out_hbm, buf0, buf1, sem0, sem1):
    c = lax.axis_index("c")
    s = lax.axis_index("s")
    b0 = s * _WIN
    bufs = (buf0, buf1)
    sems = (sem0, sem1)

    @pl.when(c == 0)
    def _():
        _half_body(x_hbm, out_hbm, bufs, sems, b0, 0)

    @pl.when(c == 1)
    def _():
        _half_body(x_hbm, out_hbm, bufs, sems, b0, _HALF)


@jax.jit
def kernel(x):
    xt = jnp.transpose(x, (1, 0, 2))
    mesh = plsc.VectorSubcoreMesh(core_axis_name="c", subcore_axis_name="s")
    out_t = pl.kernel(
        _body,
        out_type=jax.ShapeDtypeStruct((_N, _B, _D), x.dtype),
        mesh=mesh,
        scratch_types=[
            pltpu.VMEM((_WIN, _D), jnp.float32),
            pltpu.VMEM((_WIN, _D), jnp.float32),
            pltpu.SemaphoreType.DMA,
            pltpu.SemaphoreType.DMA,
        ],
    )(xt)
    return jnp.transpose(out_t, (1, 0, 2))
